# baseline (speedup 1.0000x reference)
import functools

import jax
import jax.numpy as jnp
from jax import lax
from jax.experimental import pallas as pl
from jax.experimental.pallas import tpu as pltpu
from jax.experimental.pallas import tpu_sc as plsc

NC, NS = 2, 16
NW = NC * NS
BATCH = 16384
DIM = 16
NF = 26
VTILE = 8
NT = 100000 // VTILE
ROWS_PER_W = BATCH // NW
CHUNK = 128
NCH = ROWS_PER_W // CHUNK

_mesh = plsc.VectorSubcoreMesh(core_axis_name="c", subcore_axis_name="s")


@functools.partial(
    pl.kernel,
    out_type=jax.ShapeDtypeStruct((BATCH, NF * DIM), jnp.float32),
    mesh=_mesh,
    scratch_types=[
        pltpu.VMEM((NF * ROWS_PER_W,), jnp.int32),
        pltpu.VMEM((CHUNK,), jnp.int32),
        pltpu.VMEM((CHUNK,), jnp.int32),
        pltpu.VMEM((CHUNK, VTILE * DIM), jnp.float32),
        pltpu.VMEM((CHUNK, VTILE * DIM), jnp.float32),
        pltpu.VMEM((CHUNK, NF * DIM), jnp.float32),
        pltpu.SemaphoreType.DMA,
        pltpu.SemaphoreType.DMA,
    ],
)
def _embed_sc(*refs):
    idx_refs = refs[:NF]
    tab_ref = refs[NF]
    out_ref = refs[NF + 1]
    idx_v, tq0, tq1, gd0, gd1, asm, sem0, sem1 = refs[NF + 2:]
    tq = (tq0, tq1)
    gd = (gd0, gd1)
    sem = (sem0, sem1)
    wid = lax.axis_index("s") * NC + lax.axis_index("c")
    base = wid * ROWS_PER_W
    for f in range(NF):
        pltpu.sync_copy(idx_refs[f].at[pl.ds(base, ROWS_PER_W)],
                        idx_v.at[pl.ds(f * ROWS_PER_W, ROWS_PER_W)])

    def _fire(c, f):
        b = f % 2
        off = f * ROWS_PER_W + c * CHUNK
        for blk in range(CHUNK // 16):
            tq[b][pl.ds(blk * 16, 16)] = (
                idx_v[pl.ds(off + blk * 16, 16)] >> 3) + f * NT
        return pltpu.async_copy(tab_ref.at[tq[b]], gd[b], sem[b])

    @pl.loop(0, NCH)
    def _chunk(c):
        cp = _fire(c, 0)
        for f in range(NF):
            nxt = _fire(c, f + 1) if f + 1 < NF else None
            cp.wait()
            off = f * ROWS_PER_W + c * CHUNK
            g = gd[f % 2]

            @pl.loop(0, CHUNK // 16)
            def _select(blk):
                sv = idx_v[pl.ds(off + blk * 16, 16)] & (VTILE - 1)
                for l in range(16):
                    i = blk * 16 + l
                    s = sv[l]
                    asm[i, pl.ds(f * DIM, DIM)] = g[i, pl.ds(s * DIM, DIM)]

            cp = nxt
        pltpu.sync_copy(asm, out_ref.at[pl.ds(base + c * CHUNK, CHUNK), :])


def kernel(f00, f01, f02, f03, f04, f05, f06, f07, f08, f09, f10, f11, f12,
           f13, f14, f15, f16, f17, f18, f19, f20, f21, f22, f23, f24, f25,
           W_f00, W_f01, W_f02, W_f03, W_f04, W_f05, W_f06, W_f07, W_f08,
           W_f09, W_f10, W_f11, W_f12, W_f13, W_f14, W_f15, W_f16, W_f17,
           W_f18, W_f19, W_f20, W_f21, W_f22, W_f23, W_f24, W_f25):
    idx = (f00, f01, f02, f03, f04, f05, f06, f07, f08, f09, f10, f11, f12,
           f13, f14, f15, f16, f17, f18, f19, f20, f21, f22, f23, f24, f25)
    tabs = (W_f00, W_f01, W_f02, W_f03, W_f04, W_f05, W_f06, W_f07, W_f08,
            W_f09, W_f10, W_f11, W_f12, W_f13, W_f14, W_f15, W_f16, W_f17,
            W_f18, W_f19, W_f20, W_f21, W_f22, W_f23, W_f24, W_f25)
    tbig = jnp.concatenate([w.reshape(NT, VTILE * DIM) for w in tabs], axis=0)
    return _embed_sc(*idx, tbig)


# speedup vs baseline: 1.3719x; 1.3719x over previous
import functools

import jax
import jax.numpy as jnp
from jax import lax
from jax.experimental import pallas as pl
from jax.experimental.pallas import tpu as pltpu
from jax.experimental.pallas import tpu_sc as plsc

NC, NS = 2, 16
NW = NC * NS
BATCH = 16384
DIM = 16
NF = 26
VTILE = 8
NT = 100000 // VTILE
ROWS_PER_W = BATCH // NW
CHUNK = 128
NCH = ROWS_PER_W // CHUNK

_mesh = plsc.VectorSubcoreMesh(core_axis_name="c", subcore_axis_name="s")


@functools.partial(
    pl.kernel,
    out_type=jax.ShapeDtypeStruct((BATCH, NF * DIM), jnp.float32),
    mesh=_mesh,
    scratch_types=[
        pltpu.VMEM((NF * ROWS_PER_W,), jnp.int32),
        pltpu.VMEM((CHUNK,), jnp.int32),
        pltpu.VMEM((CHUNK,), jnp.int32),
        pltpu.VMEM((CHUNK, VTILE * DIM), jnp.float32),
        pltpu.VMEM((CHUNK, VTILE * DIM), jnp.float32),
        pltpu.VMEM((CHUNK, NF * DIM), jnp.float32),
        pltpu.SemaphoreType.DMA,
        pltpu.SemaphoreType.DMA,
    ],
)
def _embed_sc(*refs):
    idx_refs = refs[:NF]
    tab_refs = refs[NF:2 * NF]
    out_ref = refs[2 * NF]
    idx_v, tq0, tq1, gd0, gd1, asm, sem0, sem1 = refs[2 * NF + 1:]
    tq = (tq0, tq1)
    gd = (gd0, gd1)
    sem = (sem0, sem1)
    wid = lax.axis_index("s") * NC + lax.axis_index("c")
    base = wid * ROWS_PER_W
    for f in range(NF):
        pltpu.sync_copy(idx_refs[f].at[pl.ds(base, ROWS_PER_W)],
                        idx_v.at[pl.ds(f * ROWS_PER_W, ROWS_PER_W)])

    def _fire(c, f):
        b = f % 2
        off = f * ROWS_PER_W + c * CHUNK
        for blk in range(CHUNK // 16):
            tq[b][pl.ds(blk * 16, 16)] = idx_v[pl.ds(off + blk * 16, 16)] >> 3
        return pltpu.async_copy(tab_refs[f].at[tq[b]], gd[b], sem[b])

    @pl.loop(0, NCH)
    def _chunk(c):
        cp = _fire(c, 0)
        for f in range(NF):
            nxt = _fire(c, f + 1) if f + 1 < NF else None
            cp.wait()
            off = f * ROWS_PER_W + c * CHUNK
            g = gd[f % 2]

            @pl.loop(0, CHUNK // 16)
            def _select(blk):
                sv = idx_v[pl.ds(off + blk * 16, 16)] & (VTILE - 1)
                for l in range(16):
                    i = blk * 16 + l
                    s = sv[l]
                    asm[i, pl.ds(f * DIM, DIM)] = g[i, pl.ds(s * DIM, DIM)]

            cp = nxt
        pltpu.sync_copy(asm, out_ref.at[pl.ds(base + c * CHUNK, CHUNK), :])


def kernel(f00, f01, f02, f03, f04, f05, f06, f07, f08, f09, f10, f11, f12,
           f13, f14, f15, f16, f17, f18, f19, f20, f21, f22, f23, f24, f25,
           W_f00, W_f01, W_f02, W_f03, W_f04, W_f05, W_f06, W_f07, W_f08,
           W_f09, W_f10, W_f11, W_f12, W_f13, W_f14, W_f15, W_f16, W_f17,
           W_f18, W_f19, W_f20, W_f21, W_f22, W_f23, W_f24, W_f25):
    idx = (f00, f01, f02, f03, f04, f05, f06, f07, f08, f09, f10, f11, f12,
           f13, f14, f15, f16, f17, f18, f19, f20, f21, f22, f23, f24, f25)
    tabs = (W_f00, W_f01, W_f02, W_f03, W_f04, W_f05, W_f06, W_f07, W_f08,
            W_f09, W_f10, W_f11, W_f12, W_f13, W_f14, W_f15, W_f16, W_f17,
            W_f18, W_f19, W_f20, W_f21, W_f22, W_f23, W_f24, W_f25)
    tabs2 = tuple(w.reshape(NT, VTILE * DIM) for w in tabs)
    return _embed_sc(*idx, *tabs2)
